# baseline (speedup 1.0000x reference)
import jax
import jax.numpy as jnp
from jax.experimental import pallas as pl
from jax.experimental.pallas import tpu as pltpu

VOCAB = 1000
BATCH_BLOCK = 128
S_PAD = 32
V_PAD = 1024


def _onehot_block(x_ref, o_ref):
    bb, s = x_ref.shape
    idx = x_ref[...].reshape(bb, s, 1)
    idx = jnp.pad(idx, ((0, 0), (0, S_PAD - s), (0, 0)), constant_values=-1)
    iota = jax.lax.broadcasted_iota(jnp.int32, (bb, S_PAD, V_PAD), 2)
    o_ref[...] = (idx == iota).astype(jnp.int8)


def kernel(x):
    B, S = x.shape
    xi = x.astype(jnp.int32)
    nblocks = B // BATCH_BLOCK
    padded = pl.pallas_call(
        _onehot_block,
        grid=(nblocks,),
        in_specs=[pl.BlockSpec((BATCH_BLOCK, S), lambda i: (i, 0))],
        out_specs=pl.BlockSpec((BATCH_BLOCK, S_PAD, V_PAD), lambda i: (i, 0, 0)),
        out_shape=jax.ShapeDtypeStruct((B, S_PAD, V_PAD), jnp.int8),
    )(xi)
    return padded[:, :S, :VOCAB].astype(jnp.float32)
